# baseline (speedup 1.0000x reference)
import functools

import jax
import jax.numpy as jnp
from jax import lax
from jax.experimental import pallas as pl
from jax.experimental.pallas import tpu as pltpu
from jax.experimental.pallas import tpu_sc as plsc

NC = 2
NS = 16
LANES = 16
CH = 128


@functools.lru_cache(maxsize=None)
def _sc_neighbor_sum(n_pad: int, d: int, nch: int, with_deg: bool):
    nw = NC * NS
    assert nch % 2 == 1 and nch >= 3
    rpt = n_pad // NS
    assert rpt * NS == n_pad
    dacc = d + LANES if with_deg else d

    mesh = plsc.VectorSubcoreMesh(
        core_axis_name="c", subcore_axis_name="s",
        num_cores=NC, num_subcores=NS)

    out_type = [jax.ShapeDtypeStruct((NC, n_pad, d), jnp.float32)]
    if with_deg:
        out_type.append(jax.ShapeDtypeStruct((NC, n_pad, LANES), jnp.float32))

    @functools.partial(
        pl.kernel,
        out_type=tuple(out_type),
        mesh=mesh,
        compiler_params=pltpu.CompilerParams(use_tc_tiling_on_sc=False),
        scratch_types=[
            pltpu.VMEM((2, CH), jnp.int32),
            pltpu.VMEM((2, CH), jnp.int32),
            pltpu.VMEM((CH, dacc), jnp.float32),
            pltpu.VMEM((CH, dacc), jnp.float32),
            pltpu.VMEM_SHARED((n_pad, dacc), jnp.float32),
            pltpu.SemaphoreType.DMA,
            pltpu.SemaphoreType.DMA,
            pltpu.SemaphoreType.DMA,
        ])
    def sc_kernel(h_hbm, eic_hbm, z_hbm, *rest):
        if with_deg:
            (sum_out, deg_out, idxA, idxB, rowsA, rowsB,
             acc_sh, semZ, semA, semB) = rest
        else:
            (sum_out, idxA, idxB, rowsA, rowsB,
             acc_sh, semZ, semA, semB) = rest
        c = lax.axis_index("c")
        s = lax.axis_index("s")
        wid = c * NS + s
        cbase = wid * nch
        tile0 = s * rpt

        zd = pltpu.async_copy(z_hbm, acc_sh.at[pl.ds(tile0, rpt)], semZ)
        pltpu.sync_copy(eic_hbm.at[cbase], idxA)
        pltpu.async_copy(h_hbm.at[idxA.at[0]], rowsA, semA)
        zd.wait()
        plsc.subcore_barrier()

        def body(k, carry):
            i = cbase + 2 * k
            pltpu.sync_copy(eic_hbm.at[i + 1], idxB)
            gB = pltpu.async_copy(h_hbm.at[idxB.at[0]], rowsB, semB)
            pltpu.make_async_copy(h_hbm.at[idxA.at[0]], rowsA, semA).wait()
            pltpu.sync_copy(rowsA, acc_sh.at[idxA.at[1]], add=True)
            pltpu.sync_copy(eic_hbm.at[i + 2], idxA)
            pltpu.async_copy(h_hbm.at[idxA.at[0]], rowsA, semA)
            gB.wait()
            pltpu.sync_copy(rowsB, acc_sh.at[idxB.at[1]], add=True)
            return carry
        lax.fori_loop(0, nch // 2, body, None)

        pltpu.make_async_copy(h_hbm.at[idxA.at[0]], rowsA, semA).wait()
        pltpu.sync_copy(rowsA, acc_sh.at[idxA.at[1]], add=True)

        plsc.subcore_barrier()

        if with_deg:
            pltpu.sync_copy(acc_sh.at[pl.ds(tile0, rpt), pl.ds(0, d)],
                            sum_out.at[c, pl.ds(tile0, rpt)])
            pltpu.sync_copy(acc_sh.at[pl.ds(tile0, rpt), pl.ds(d, LANES)],
                            deg_out.at[c, pl.ds(tile0, rpt)])
        else:
            pltpu.sync_copy(acc_sh.at[pl.ds(tile0, rpt)],
                            sum_out.at[c, pl.ds(tile0, rpt)])

    return sc_kernel


@functools.lru_cache(maxsize=None)
def _ones_col(n: int, d: int):
    bm = 1024

    def body(x_ref, o_ref):
        lane = lax.broadcasted_iota(jnp.int32, (bm, LANES), 1)
        tail = jnp.where(lane == 0, 1.0, 0.0).astype(jnp.float32)
        o_ref[...] = jnp.concatenate([x_ref[...], tail], axis=1)

    return pl.pallas_call(
        body,
        grid=(-(-n // bm),),
        in_specs=[pl.BlockSpec((bm, d), lambda i: (i, 0))],
        out_specs=pl.BlockSpec((bm, d + LANES), lambda i: (i, 0)),
        out_shape=jax.ShapeDtypeStruct((n, d + LANES), jnp.float32),
    )


@functools.lru_cache(maxsize=None)
def _tc_layer(n: int, n_pad: int, d: int):
    bm = 1024
    assert n_pad % bm == 0

    def body(h_ref, p_ref, deg_ref, ws_ref, wn_ref, o_ref):
        deg = jnp.maximum(deg_ref[0, :, :1] + deg_ref[1, :, :1], 1.0)
        m = (p_ref[0] + p_ref[1]) / deg
        dn = (((1,), (1,)), ((), ()))
        acc = lax.dot_general(h_ref[...], ws_ref[...], dn,
                              preferred_element_type=jnp.float32)
        acc = acc + lax.dot_general(m, wn_ref[...], dn,
                                    preferred_element_type=jnp.float32)
        o_ref[...] = jnp.maximum(acc, 0.0)

    return pl.pallas_call(
        body,
        grid=(n_pad // bm,),
        in_specs=[
            pl.BlockSpec((bm, d), lambda i: (i, 0)),
            pl.BlockSpec((NC, bm, d), lambda i: (0, i, 0)),
            pl.BlockSpec((NC, bm, LANES), lambda i: (0, i, 0)),
            pl.BlockSpec((d, d), lambda i: (0, 0)),
            pl.BlockSpec((d, d), lambda i: (0, 0)),
        ],
        out_specs=pl.BlockSpec((bm, d), lambda i: (i, 0)),
        out_shape=jax.ShapeDtypeStruct((n, d), jnp.float32),
    )


def kernel(x, weight, edge_index):
    n, d = x.shape
    e = edge_index.shape[1]
    n_pad = -(-n // 2048) * 2048
    rpt = n_pad // NS
    nw = NC * NS

    nch = -(-e // (nw * CH))
    if nch % 2 == 0:
        nch += 1
    e_tot = nw * nch * CH
    pad = e_tot - e
    src_p = jnp.concatenate([edge_index[0], jnp.zeros((pad,), jnp.int32)])
    dst_p = jnp.concatenate([edge_index[1],
                             jnp.full((pad,), n_pad - 1, jnp.int32)])
    eic = jnp.stack([src_p.reshape(-1, CH), dst_p.reshape(-1, CH)], axis=1)

    zw = jnp.zeros((rpt, d + LANES), jnp.float32)
    zn = jnp.zeros((rpt, d), jnp.float32)
    h0w = _ones_col(n, d)(x)
    feat1, degp = _sc_neighbor_sum(n_pad, d, nch, True)(h0w, eic, zw)
    tc = _tc_layer(n, n_pad, d)
    h1 = tc(x, feat1, degp, weight[0, :, :d], weight[0, :, d:])
    (feat2,) = _sc_neighbor_sum(n_pad, d, nch, False)(h1, eic, zn)
    h2 = tc(h1, feat2, degp, weight[1, :, :d], weight[1, :, d:])
    return h2


# speedup vs baseline: 1.6968x; 1.6968x over previous
import functools

import jax
import jax.numpy as jnp
from jax import lax
from jax.experimental import pallas as pl
from jax.experimental.pallas import tpu as pltpu
from jax.experimental.pallas import tpu_sc as plsc

NC = 2
NS = 16
LANES = 16
CH = 80


@functools.lru_cache(maxsize=None)
def _sc_neighbor_sum(n_pad: int, d: int, nch: int, with_deg: bool):
    nw = NC * NS
    assert nch % 2 == 1 and nch >= 3
    rpt = n_pad // NS
    assert rpt * NS == n_pad
    dacc = d + LANES if with_deg else d

    mesh = plsc.VectorSubcoreMesh(
        core_axis_name="c", subcore_axis_name="s",
        num_cores=NC, num_subcores=NS)

    out_type = [jax.ShapeDtypeStruct((NC, n_pad, d), jnp.float32)]
    if with_deg:
        out_type.append(jax.ShapeDtypeStruct((NC, n_pad, LANES), jnp.float32))

    @functools.partial(
        pl.kernel,
        out_type=tuple(out_type),
        mesh=mesh,
        compiler_params=pltpu.CompilerParams(use_tc_tiling_on_sc=False),
        scratch_types=[
            pltpu.VMEM((2, CH), jnp.int32),
            pltpu.VMEM((2, CH), jnp.int32),
            pltpu.VMEM((CH, dacc), jnp.float32),
            pltpu.VMEM((CH, dacc), jnp.float32),
            pltpu.VMEM_SHARED((n_pad, dacc), jnp.float32),
            pltpu.SemaphoreType.DMA,
            pltpu.SemaphoreType.DMA,
            pltpu.SemaphoreType.DMA,
        ])
    def sc_kernel(h_hbm, eic_hbm, z_hbm, *rest):
        if with_deg:
            (sum_out, deg_out, idxA, idxB, rowsA, rowsB,
             acc_sh, semZ, semA, semB) = rest
        else:
            (sum_out, idxA, idxB, rowsA, rowsB,
             acc_sh, semZ, semA, semB) = rest
        c = lax.axis_index("c")
        s = lax.axis_index("s")
        wid = c * NS + s
        cbase = wid * nch
        tile0 = s * rpt

        zd = pltpu.async_copy(z_hbm, acc_sh.at[pl.ds(tile0, rpt)], semZ)
        pltpu.sync_copy(eic_hbm.at[cbase], idxA)
        pltpu.async_copy(h_hbm.at[idxA.at[0]], rowsA, semA)
        zd.wait()
        plsc.subcore_barrier()

        def body(k, carry):
            i = cbase + 2 * k
            pltpu.sync_copy(eic_hbm.at[i + 1], idxB)
            gB = pltpu.async_copy(h_hbm.at[idxB.at[0]], rowsB, semB)
            pltpu.make_async_copy(h_hbm.at[idxA.at[0]], rowsA, semA).wait()
            pltpu.sync_copy(rowsA, acc_sh.at[idxA.at[1]], add=True)
            pltpu.sync_copy(eic_hbm.at[i + 2], idxA)
            pltpu.async_copy(h_hbm.at[idxA.at[0]], rowsA, semA)
            gB.wait()
            pltpu.sync_copy(rowsB, acc_sh.at[idxB.at[1]], add=True)
            return carry
        lax.fori_loop(0, nch // 2, body, None)

        pltpu.make_async_copy(h_hbm.at[idxA.at[0]], rowsA, semA).wait()
        pltpu.sync_copy(rowsA, acc_sh.at[idxA.at[1]], add=True)

        plsc.subcore_barrier()

        if with_deg:
            pltpu.sync_copy(acc_sh.at[pl.ds(tile0, rpt), pl.ds(0, d)],
                            sum_out.at[c, pl.ds(tile0, rpt)])
            pltpu.sync_copy(acc_sh.at[pl.ds(tile0, rpt), pl.ds(d, LANES)],
                            deg_out.at[c, pl.ds(tile0, rpt)])
        else:
            pltpu.sync_copy(acc_sh.at[pl.ds(tile0, rpt)],
                            sum_out.at[c, pl.ds(tile0, rpt)])

    return sc_kernel


@functools.lru_cache(maxsize=None)
def _ones_col(n: int, d: int):
    bm = 1024

    def body(x_ref, o_ref):
        lane = lax.broadcasted_iota(jnp.int32, (bm, LANES), 1)
        tail = jnp.where(lane == 0, 1.0, 0.0).astype(jnp.float32)
        o_ref[...] = jnp.concatenate([x_ref[...], tail], axis=1)

    return pl.pallas_call(
        body,
        grid=(-(-n // bm),),
        in_specs=[pl.BlockSpec((bm, d), lambda i: (i, 0))],
        out_specs=pl.BlockSpec((bm, d + LANES), lambda i: (i, 0)),
        out_shape=jax.ShapeDtypeStruct((n, d + LANES), jnp.float32),
    )


@functools.lru_cache(maxsize=None)
def _tc_layer(n: int, n_pad: int, d: int):
    bm = 1024
    assert n_pad % bm == 0

    def body(h_ref, p_ref, deg_ref, ws_ref, wn_ref, o_ref):
        deg = jnp.maximum(deg_ref[0, :, :1] + deg_ref[1, :, :1], 1.0)
        m = (p_ref[0] + p_ref[1]) / deg
        dn = (((1,), (1,)), ((), ()))
        acc = lax.dot_general(h_ref[...], ws_ref[...], dn,
                              preferred_element_type=jnp.float32)
        acc = acc + lax.dot_general(m, wn_ref[...], dn,
                                    preferred_element_type=jnp.float32)
        o_ref[...] = jnp.maximum(acc, 0.0)

    return pl.pallas_call(
        body,
        grid=(n_pad // bm,),
        in_specs=[
            pl.BlockSpec((bm, d), lambda i: (i, 0)),
            pl.BlockSpec((NC, bm, d), lambda i: (0, i, 0)),
            pl.BlockSpec((NC, bm, LANES), lambda i: (0, i, 0)),
            pl.BlockSpec((d, d), lambda i: (0, 0)),
            pl.BlockSpec((d, d), lambda i: (0, 0)),
        ],
        out_specs=pl.BlockSpec((bm, d), lambda i: (i, 0)),
        out_shape=jax.ShapeDtypeStruct((n, d), jnp.float32),
    )


def kernel(x, weight, edge_index):
    n, d = x.shape
    e = edge_index.shape[1]
    n_pad = -(-n // 2048) * 2048
    rpt = n_pad // NS
    nw = NC * NS

    nch = -(-e // (nw * CH))
    if nch % 2 == 0:
        nch += 1
    e_tot = nw * nch * CH
    pad = e_tot - e
    src_p = jnp.concatenate([edge_index[0], jnp.zeros((pad,), jnp.int32)])
    dst_p = jnp.concatenate([edge_index[1],
                             jnp.full((pad,), n_pad - 1, jnp.int32)])
    eic = jnp.stack([src_p.reshape(-1, CH), dst_p.reshape(-1, CH)], axis=1)

    zw = jnp.zeros((rpt, d + LANES), jnp.float32)
    zn = jnp.zeros((rpt, d), jnp.float32)
    h0w = _ones_col(n, d)(x)
    feat1, degp = _sc_neighbor_sum(n_pad, d, nch, True)(h0w, eic, zw)
    tc = _tc_layer(n, n_pad, d)
    h1 = tc(x, feat1, degp, weight[0, :, :d], weight[0, :, d:])
    (feat2,) = _sc_neighbor_sum(n_pad, d, nch, False)(h1, eic, zn)
    h2 = tc(h1, feat2, degp, weight[1, :, :d], weight[1, :, d:])
    return h2
